# baseline (speedup 1.0000x reference)
import jax
import jax.numpy as jnp
from jax import lax
from jax.experimental import pallas as pl
from jax.experimental.pallas import tpu as pltpu
from jax.experimental.pallas import tpu_sc as plsc

BATCH = 16384
EMB_DIM = 32
PACK = 4
NUM_CORES = 2
NUM_SUBCORES = 16
NUM_WORKERS = NUM_CORES * NUM_SUBCORES
B_PER_W = BATCH // NUM_WORKERS
LANES = 16
CHUNK = 128
NCHUNKS = B_PER_W // CHUNK
GROUPS = CHUNK // LANES


def _dot_kernel(urow_hbm, uoff_hbm, irow_hbm, ioff_hbm, uemb_hbm, iemb_hbm,
                out_hbm, urow_v, uoff_v, irow_v, ioff_v,
                ubuf0, ubuf1, ibuf0, ibuf1, out_v,
                sem_u0, sem_u1, sem_i0, sem_i1):
    wid = lax.axis_index("s") * NUM_CORES + lax.axis_index("c")
    base = wid * B_PER_W

    pltpu.sync_copy(urow_hbm.at[pl.ds(base, B_PER_W)], urow_v)
    pltpu.sync_copy(irow_hbm.at[pl.ds(base, B_PER_W)], irow_v)
    pltpu.sync_copy(uoff_hbm.at[pl.ds(base, B_PER_W)], uoff_v)
    pltpu.sync_copy(ioff_hbm.at[pl.ds(base, B_PER_W)], ioff_v)

    ubufs = (ubuf0, ubuf1)
    ibufs = (ibuf0, ibuf1)
    usems = (sem_u0, sem_u1)
    isems = (sem_i0, sem_i1)

    def start_gathers(c):
        p = c % 2
        cu = pltpu.async_copy(
            uemb_hbm.at[urow_v.at[pl.ds(c * CHUNK, CHUNK)]], ubufs[p], usems[p])
        ci = pltpu.async_copy(
            iemb_hbm.at[irow_v.at[pl.ds(c * CHUNK, CHUNK)]], ibufs[p], isems[p])
        return cu, ci

    iota16 = lax.iota(jnp.int32, LANES)
    pending = start_gathers(0)

    for c in range(NCHUNKS):
        p = c % 2
        ub, ib = ubufs[p], ibufs[p]
        cu, ci = pending
        if c + 1 < NCHUNKS:
            nxt = start_gathers(c + 1)
        cu.wait()
        ci.wait()
        if c + 1 < NCHUNKS:
            pending = nxt

        @pl.loop(0, GROUPS)
        def _(g):
            row_ids = g * LANES + iota16
            acc = jnp.zeros((LANES,), jnp.float32)
            off_u = uoff_v[pl.ds(c * CHUNK + g * LANES, LANES)]
            off_i = ioff_v[pl.ds(c * CHUNK + g * LANES, LANES)]
            for d in range(EMB_DIM):
                u = plsc.load_gather(ub, [row_ids, off_u + d])
                v = plsc.load_gather(ib, [row_ids, off_i + d])
                acc = acc + u * v
            out_v[pl.ds(c * CHUNK + g * LANES, LANES)] = acc

    pltpu.sync_copy(out_v, out_hbm.at[pl.ds(base, B_PER_W)])


@jax.jit
def kernel(user, item, user_emb, item_emb):
    user = user.astype(jnp.int32)
    item = item.astype(jnp.int32)
    urow = user // PACK
    uoff = (user % PACK) * EMB_DIM
    irow = item // PACK
    ioff = (item % PACK) * EMB_DIM
    uemb4 = user_emb.reshape(user_emb.shape[0] // PACK, EMB_DIM * PACK)
    iemb4 = item_emb.reshape(item_emb.shape[0] // PACK, EMB_DIM * PACK)

    mesh = plsc.VectorSubcoreMesh(core_axis_name="c", subcore_axis_name="s")
    run = pl.kernel(
        _dot_kernel,
        out_type=jax.ShapeDtypeStruct((BATCH,), jnp.float32),
        mesh=mesh,
        compiler_params=pltpu.CompilerParams(needs_layout_passes=False),
        scratch_types=[
            pltpu.VMEM((B_PER_W,), jnp.int32),
            pltpu.VMEM((B_PER_W,), jnp.int32),
            pltpu.VMEM((B_PER_W,), jnp.int32),
            pltpu.VMEM((B_PER_W,), jnp.int32),
            pltpu.VMEM((CHUNK, EMB_DIM * PACK), jnp.float32),
            pltpu.VMEM((CHUNK, EMB_DIM * PACK), jnp.float32),
            pltpu.VMEM((CHUNK, EMB_DIM * PACK), jnp.float32),
            pltpu.VMEM((CHUNK, EMB_DIM * PACK), jnp.float32),
            pltpu.VMEM((B_PER_W,), jnp.float32),
            pltpu.SemaphoreType.DMA,
            pltpu.SemaphoreType.DMA,
            pltpu.SemaphoreType.DMA,
            pltpu.SemaphoreType.DMA,
        ],
    )
    return run(urow, uoff, irow, ioff, uemb4, iemb4)
